# baseline (speedup 1.0000x reference)
import jax
import jax.numpy as jnp
from jax import lax
from jax.experimental import pallas as pl
from jax.experimental.pallas import tpu as pltpu
from jax.experimental.pallas import tpu_sc as plsc

N = 500000
NCAT = 1000
D = 64
DP = 128
TP = 512
H = 32

NC = 2
NS = 16
NW = NC * NS
L = 16

NPAD = 524288
NPAD2 = NPAD // 2
CPW = NPAD2 // NW
BS_SC = 128
NIT = CPW // BS_SC
NB = 2

BT = 4096
GRID_A = NPAD2 // BT
GRID_B = (N + BT - 1) // BT - GRID_A


def _prep_body(embe_ref, embo_ref, wcaT_ref, wcbT_ref, wcb_ref, w2_ref,
               b2_ref, bc_ref, t2_ref, mT_ref):
    wcaT = wcaT_ref[...]
    const = jnp.dot(b2_ref[...], wcbT_ref[...], preferred_element_type=jnp.float32)
    const = const + bc_ref[...]
    left = jnp.dot(embe_ref[...], wcaT, preferred_element_type=jnp.float32) + const
    right = jnp.dot(embo_ref[...], wcaT, preferred_element_type=jnp.float32) + const
    paired = jnp.concatenate([left, right], axis=1)
    t2_ref[...] = jnp.concatenate(
        [paired, jnp.zeros((TP - NCAT // 2, DP), jnp.float32)], axis=0)
    mT_ref[...] = jnp.dot(wcb_ref[...], w2_ref[...],
                          preferred_element_type=jnp.float32)


def _sc_gather_body(idx_hbm, table_hbm, out_hbm, t2v, idx_v0, idx_v1,
                    ob0, ob1, sem_i0, sem_i1, sem_o0, sem_o1):
    wid = lax.axis_index("s") * NC + lax.axis_index("c")
    base = wid * CPW
    idx_v = [idx_v0, idx_v1]
    ob = [ob0, ob1]
    sem_i = [sem_i0, sem_i1]
    sem_o = [sem_o0, sem_o1]

    pltpu.sync_copy(table_hbm, t2v)

    for b in range(NB):
        pltpu.async_copy(idx_hbm.at[pl.ds(base + b * BS_SC, BS_SC)],
                         idx_v[b], sem_i[b])

    lane = lax.iota(jnp.int32, L)

    def body(jo, carry):
        for b in range(NB):
            j = jo * NB + b
            off = base + j * BS_SC
            pltpu.make_async_copy(idx_hbm.at[pl.ds(off, BS_SC)],
                                  idx_v[b], sem_i[b]).wait()
            @pl.when(j >= NB)
            def _wait_prev():
                pltpu.make_async_copy(
                    ob[b],
                    out_hbm.at[:, pl.ds(off - NB * BS_SC, BS_SC)],
                    sem_o[b]).wait()
            obb = ob[b]
            idxb = idx_v[b]

            def tbody(t, c):
                cat = idxb[pl.ds(t * L, L)]
                prow = lax.shift_right_logical(cat, 1)
                pbase = lax.shift_left(lax.bitwise_and(cat, 1), 6)
                srow = t * L + lane
                for c0 in range(0, D, 4):
                    xrows = [lax.bitwise_xor(lane, c0 + i) for i in range(4)]
                    vs = [plsc.load_gather(t2v, [prow, pbase + xrows[i]])
                          for i in range(4)]
                    for i in range(4):
                        plsc.store_scatter(obb, [xrows[i], srow], vs[i])
                return c

            lax.fori_loop(0, BS_SC // L, tbody, 0)
            @pl.when(j + NB < NIT)
            def _prefetch():
                pltpu.async_copy(idx_hbm.at[pl.ds(off + NB * BS_SC, BS_SC)],
                                 idx_v[b], sem_i[b])
            pltpu.async_copy(ob[b], out_hbm.at[:, pl.ds(off, BS_SC)], sem_o[b])
        return carry

    lax.fori_loop(0, NIT // NB, body, 0)

    for b in range(NB):
        off = base + (NIT - NB + b) * BS_SC
        pltpu.make_async_copy(ob[b], out_hbm.at[:, pl.ds(off, BS_SC)],
                              sem_o[b]).wait()


def _combine_body(g_ref, s_ref, w15_ref, mT_ref, o_ref):
    h = jnp.dot(w15_ref[...], s_ref[...], preferred_element_type=jnp.float32)
    h = jnp.maximum(h, 0.0)
    p = jnp.dot(mT_ref[...], h, preferred_element_type=jnp.float32)
    o_ref[...] = g_ref[...] + p


def _combine_body_b(prev_ref, g_ref, s_ref, w15_ref, mT_ref, o_ref):
    del prev_ref
    h = jnp.dot(w15_ref[...], s_ref[...], preferred_element_type=jnp.float32)
    h = jnp.maximum(h, 0.0)
    p = jnp.dot(mT_ref[...], h, preferred_element_type=jnp.float32)
    o_ref[...] = g_ref[...] + p


def kernel(categories, popularity, cluster_ids, temporal_scores,
           geographic_scores, emb_table, W1, b1, W2, b2, Wc, bc):
    cats = categories.astype(jnp.int32)
    cats_pad = jnp.concatenate([cats, jnp.zeros((NPAD - N,), jnp.int32)])
    ones = jnp.ones((N,), jnp.float32)
    stats5 = jnp.stack([popularity, cluster_ids, temporal_scores,
                        geographic_scores, ones], axis=0)

    wcaT = Wc[:, :H].T
    wcbT = Wc[:, H:].T
    wcb = Wc[:, H:]
    b2r = b2.reshape(1, H)
    bcr = bc.reshape(1, D)
    w15 = jnp.concatenate([W1, b1.reshape(H, 1)], axis=1)

    t2p, mT = pl.pallas_call(
        _prep_body,
        out_shape=(
            jax.ShapeDtypeStruct((TP, DP), jnp.float32),
            jax.ShapeDtypeStruct((D, H), jnp.float32),
        ),
    )(emb_table[0::2], emb_table[1::2], wcaT, wcbT, wcb, W2, b2r, bcr)

    mesh = plsc.VectorSubcoreMesh(core_axis_name="c", subcore_axis_name="s")
    sc_kwargs = dict(
        mesh=mesh,
        out_type=jax.ShapeDtypeStruct((D, NPAD2), jnp.float32),
        scratch_types=[
            pltpu.VMEM((TP, DP), jnp.float32),
            pltpu.VMEM((BS_SC,), jnp.int32),
            pltpu.VMEM((BS_SC,), jnp.int32),
            pltpu.VMEM((D, BS_SC), jnp.float32),
            pltpu.VMEM((D, BS_SC), jnp.float32),
            pltpu.SemaphoreType.DMA,
            pltpu.SemaphoreType.DMA,
            pltpu.SemaphoreType.DMA,
            pltpu.SemaphoreType.DMA,
        ],
        compiler_params=pltpu.CompilerParams(use_tc_tiling_on_sc=True,
                                             needs_layout_passes=False),
    )
    g_ta = pl.kernel(_sc_gather_body, **sc_kwargs)(cats_pad[:NPAD2], t2p)
    cats_b, _ = lax.optimization_barrier((cats_pad[NPAD2:], g_ta))
    g_tb = pl.kernel(_sc_gather_body, **sc_kwargs)(cats_b, t2p)

    out_a = pl.pallas_call(
        _combine_body,
        grid=(GRID_A,),
        in_specs=[
            pl.BlockSpec((D, BT), lambda i: (0, i)),
            pl.BlockSpec((5, BT), lambda i: (0, i)),
            pl.BlockSpec((H, 5), lambda i: (0, 0)),
            pl.BlockSpec((D, H), lambda i: (0, 0)),
        ],
        out_specs=pl.BlockSpec((D, BT), lambda i: (0, i)),
        out_shape=jax.ShapeDtypeStruct((D, N), jnp.float32),
    )(g_ta, stats5, w15, mT)

    out_t = pl.pallas_call(
        _combine_body_b,
        grid=(GRID_B,),
        in_specs=[
            pl.BlockSpec(memory_space=pl.ANY),
            pl.BlockSpec((D, BT), lambda i: (0, i)),
            pl.BlockSpec((5, BT), lambda i: (0, i + GRID_A)),
            pl.BlockSpec((H, 5), lambda i: (0, 0)),
            pl.BlockSpec((D, H), lambda i: (0, 0)),
        ],
        out_specs=pl.BlockSpec((D, BT), lambda i: (0, i + GRID_A)),
        out_shape=jax.ShapeDtypeStruct((D, N), jnp.float32),
        input_output_aliases={0: 0},
    )(out_a, g_tb, stats5, w15, mT)
    return out_t.T


# speedup vs baseline: 1.3446x; 1.3446x over previous
import jax
import jax.numpy as jnp
from jax import lax
from jax.experimental import pallas as pl
from jax.experimental.pallas import tpu as pltpu
from jax.experimental.pallas import tpu_sc as plsc

N = 500000
NCAT = 1000
D = 64
DP = 128
TP = 512
H = 32

NC = 2
NS = 16
NW = NC * NS
L = 16

NPAD = 524288
NPAD2 = NPAD // 2
CPW = NPAD2 // NW
BS_SC = 128
NIT = CPW // BS_SC
NB = 2

BT = 8192
GRID_A = NPAD2 // BT
GRID_B = (N + BT - 1) // BT - GRID_A


def _prep_body(embe_ref, embo_ref, wcaT_ref, wcbT_ref, wcb_ref, w2_ref,
               b2_ref, bc_ref, t2_ref, mT_ref):
    wcaT = wcaT_ref[...]
    const = jnp.dot(b2_ref[...], wcbT_ref[...], preferred_element_type=jnp.float32)
    const = const + bc_ref[...]
    left = jnp.dot(embe_ref[...], wcaT, preferred_element_type=jnp.float32) + const
    right = jnp.dot(embo_ref[...], wcaT, preferred_element_type=jnp.float32) + const
    paired = jnp.concatenate([left, right], axis=1)
    t2_ref[...] = jnp.concatenate(
        [paired, jnp.zeros((TP - NCAT // 2, DP), jnp.float32)], axis=0)
    mT_ref[...] = jnp.dot(wcb_ref[...], w2_ref[...],
                          preferred_element_type=jnp.float32)


def _sc_gather_body(idx_hbm, table_hbm, out_hbm, t2v, idx_v0, idx_v1,
                    ob0, ob1, sem_i0, sem_i1, sem_o0, sem_o1):
    wid = lax.axis_index("s") * NC + lax.axis_index("c")
    base = wid * CPW
    idx_v = [idx_v0, idx_v1]
    ob = [ob0, ob1]
    sem_i = [sem_i0, sem_i1]
    sem_o = [sem_o0, sem_o1]

    pltpu.sync_copy(table_hbm, t2v)

    for b in range(NB):
        pltpu.async_copy(idx_hbm.at[pl.ds(base + b * BS_SC, BS_SC)],
                         idx_v[b], sem_i[b])

    lane = lax.iota(jnp.int32, L)

    def body(jo, carry):
        for b in range(NB):
            j = jo * NB + b
            off = base + j * BS_SC
            pltpu.make_async_copy(idx_hbm.at[pl.ds(off, BS_SC)],
                                  idx_v[b], sem_i[b]).wait()
            @pl.when(j >= NB)
            def _wait_prev():
                pltpu.make_async_copy(
                    ob[b],
                    out_hbm.at[:, pl.ds(off - NB * BS_SC, BS_SC)],
                    sem_o[b]).wait()
            obb = ob[b]
            idxb = idx_v[b]

            def tbody(t, c):
                cat = idxb[pl.ds(t * L, L)]
                prow = lax.shift_right_logical(cat, 1)
                pbase = lax.shift_left(lax.bitwise_and(cat, 1), 6)
                srow = t * L + lane
                for c0 in range(0, D, 4):
                    xrows = [lax.bitwise_xor(lane, c0 + i) for i in range(4)]
                    vs = [plsc.load_gather(t2v, [prow, pbase + xrows[i]])
                          for i in range(4)]
                    for i in range(4):
                        plsc.store_scatter(obb, [xrows[i], srow], vs[i])
                return c

            lax.fori_loop(0, BS_SC // L, tbody, 0)
            @pl.when(j + NB < NIT)
            def _prefetch():
                pltpu.async_copy(idx_hbm.at[pl.ds(off + NB * BS_SC, BS_SC)],
                                 idx_v[b], sem_i[b])
            pltpu.async_copy(ob[b], out_hbm.at[:, pl.ds(off, BS_SC)], sem_o[b])
        return carry

    lax.fori_loop(0, NIT // NB, body, 0)

    for b in range(NB):
        off = base + (NIT - NB + b) * BS_SC
        pltpu.make_async_copy(ob[b], out_hbm.at[:, pl.ds(off, BS_SC)],
                              sem_o[b]).wait()


def _combine_body(g_ref, s_ref, w15_ref, mT_ref, o_ref):
    h = jnp.dot(w15_ref[...], s_ref[...], preferred_element_type=jnp.float32)
    h = jnp.maximum(h, 0.0)
    p = jnp.dot(mT_ref[...], h, preferred_element_type=jnp.float32)
    o_ref[...] = g_ref[...] + p


def _combine_body_b(prev_ref, g_ref, s_ref, w15_ref, mT_ref, o_ref):
    del prev_ref
    h = jnp.dot(w15_ref[...], s_ref[...], preferred_element_type=jnp.float32)
    h = jnp.maximum(h, 0.0)
    p = jnp.dot(mT_ref[...], h, preferred_element_type=jnp.float32)
    o_ref[...] = g_ref[...] + p


def kernel(categories, popularity, cluster_ids, temporal_scores,
           geographic_scores, emb_table, W1, b1, W2, b2, Wc, bc):
    cats = categories.astype(jnp.int32)
    cats_pad = jnp.concatenate([cats, jnp.zeros((NPAD - N,), jnp.int32)])
    ones = jnp.ones((N,), jnp.float32)
    stats5 = jnp.stack([popularity, cluster_ids, temporal_scores,
                        geographic_scores, ones], axis=0)

    wcaT = Wc[:, :H].T
    wcbT = Wc[:, H:].T
    wcb = Wc[:, H:]
    b2r = b2.reshape(1, H)
    bcr = bc.reshape(1, D)
    w15 = jnp.concatenate([W1, b1.reshape(H, 1)], axis=1)

    t2p, mT = pl.pallas_call(
        _prep_body,
        out_shape=(
            jax.ShapeDtypeStruct((TP, DP), jnp.float32),
            jax.ShapeDtypeStruct((D, H), jnp.float32),
        ),
    )(emb_table[0::2], emb_table[1::2], wcaT, wcbT, wcb, W2, b2r, bcr)

    mesh = plsc.VectorSubcoreMesh(core_axis_name="c", subcore_axis_name="s")
    sc_kwargs = dict(
        mesh=mesh,
        out_type=jax.ShapeDtypeStruct((D, NPAD2), jnp.float32),
        scratch_types=[
            pltpu.VMEM((TP, DP), jnp.float32),
            pltpu.VMEM((BS_SC,), jnp.int32),
            pltpu.VMEM((BS_SC,), jnp.int32),
            pltpu.VMEM((D, BS_SC), jnp.float32),
            pltpu.VMEM((D, BS_SC), jnp.float32),
            pltpu.SemaphoreType.DMA,
            pltpu.SemaphoreType.DMA,
            pltpu.SemaphoreType.DMA,
            pltpu.SemaphoreType.DMA,
        ],
        compiler_params=pltpu.CompilerParams(use_tc_tiling_on_sc=True,
                                             needs_layout_passes=False),
    )
    g_ta = pl.kernel(_sc_gather_body, **sc_kwargs)(cats_pad[:NPAD2], t2p)
    g_tb = pl.kernel(_sc_gather_body, **sc_kwargs)(cats_pad[NPAD2:], t2p)

    out_a = pl.pallas_call(
        _combine_body,
        grid=(GRID_A,),
        in_specs=[
            pl.BlockSpec((D, BT), lambda i: (0, i)),
            pl.BlockSpec((5, BT), lambda i: (0, i)),
            pl.BlockSpec((H, 5), lambda i: (0, 0)),
            pl.BlockSpec((D, H), lambda i: (0, 0)),
        ],
        out_specs=pl.BlockSpec((D, BT), lambda i: (0, i)),
        out_shape=jax.ShapeDtypeStruct((D, N), jnp.float32),
    )(g_ta, stats5, w15, mT)

    out_t = pl.pallas_call(
        _combine_body_b,
        grid=(GRID_B,),
        in_specs=[
            pl.BlockSpec(memory_space=pl.ANY),
            pl.BlockSpec((D, BT), lambda i: (0, i)),
            pl.BlockSpec((5, BT), lambda i: (0, i + GRID_A)),
            pl.BlockSpec((H, 5), lambda i: (0, 0)),
            pl.BlockSpec((D, H), lambda i: (0, 0)),
        ],
        out_specs=pl.BlockSpec((D, BT), lambda i: (0, i + GRID_A)),
        out_shape=jax.ShapeDtypeStruct((D, N), jnp.float32),
        input_output_aliases={0: 0},
    )(out_a, g_tb, stats5, w15, mT)
    return out_t.T


# speedup vs baseline: 1.3508x; 1.0046x over previous
import jax
import jax.numpy as jnp
from jax import lax
from jax.experimental import pallas as pl
from jax.experimental.pallas import tpu as pltpu
from jax.experimental.pallas import tpu_sc as plsc

N = 500000
NCAT = 1000
D = 64
DP = 128
TP = 512
H = 32

NC = 2
NS = 16
NW = NC * NS
L = 16

NPAD = 524288
NPAD2 = NPAD // 2
CPW = NPAD2 // NW
BS_SC = 128
NIT = CPW // BS_SC
NB = 2

BT = 16384
GRID_A = NPAD2 // BT
GRID_B = (N + BT - 1) // BT - GRID_A


def _prep_body(embe_ref, embo_ref, wcaT_ref, wcbT_ref, wcb_ref, w2_ref,
               b2_ref, bc_ref, t2_ref, mT_ref):
    wcaT = wcaT_ref[...]
    const = jnp.dot(b2_ref[...], wcbT_ref[...], preferred_element_type=jnp.float32)
    const = const + bc_ref[...]
    left = jnp.dot(embe_ref[...], wcaT, preferred_element_type=jnp.float32) + const
    right = jnp.dot(embo_ref[...], wcaT, preferred_element_type=jnp.float32) + const
    paired = jnp.concatenate([left, right], axis=1)
    t2_ref[...] = jnp.concatenate(
        [paired, jnp.zeros((TP - NCAT // 2, DP), jnp.float32)], axis=0)
    mT_ref[...] = jnp.dot(wcb_ref[...], w2_ref[...],
                          preferred_element_type=jnp.float32)


def _sc_gather_body(idx_hbm, table_hbm, out_hbm, t2v, idx_v0, idx_v1,
                    ob0, ob1, sem_i0, sem_i1, sem_o0, sem_o1):
    wid = lax.axis_index("s") * NC + lax.axis_index("c")
    base = wid * CPW
    idx_v = [idx_v0, idx_v1]
    ob = [ob0, ob1]
    sem_i = [sem_i0, sem_i1]
    sem_o = [sem_o0, sem_o1]

    pltpu.sync_copy(table_hbm, t2v)

    for b in range(NB):
        pltpu.async_copy(idx_hbm.at[pl.ds(base + b * BS_SC, BS_SC)],
                         idx_v[b], sem_i[b])

    lane = lax.iota(jnp.int32, L)

    def body(jo, carry):
        for b in range(NB):
            j = jo * NB + b
            off = base + j * BS_SC
            pltpu.make_async_copy(idx_hbm.at[pl.ds(off, BS_SC)],
                                  idx_v[b], sem_i[b]).wait()
            @pl.when(j >= NB)
            def _wait_prev():
                pltpu.make_async_copy(
                    ob[b],
                    out_hbm.at[:, pl.ds(off - NB * BS_SC, BS_SC)],
                    sem_o[b]).wait()
            obb = ob[b]
            idxb = idx_v[b]

            def tbody(t, c):
                cat = idxb[pl.ds(t * L, L)]
                prow = lax.shift_right_logical(cat, 1)
                pbase = lax.shift_left(lax.bitwise_and(cat, 1), 6)
                srow = t * L + lane
                for c0 in range(0, D, 4):
                    xrows = [lax.bitwise_xor(lane, c0 + i) for i in range(4)]
                    vs = [plsc.load_gather(t2v, [prow, pbase + xrows[i]])
                          for i in range(4)]
                    for i in range(4):
                        plsc.store_scatter(obb, [xrows[i], srow], vs[i])
                return c

            lax.fori_loop(0, BS_SC // L, tbody, 0)
            @pl.when(j + NB < NIT)
            def _prefetch():
                pltpu.async_copy(idx_hbm.at[pl.ds(off + NB * BS_SC, BS_SC)],
                                 idx_v[b], sem_i[b])
            pltpu.async_copy(ob[b], out_hbm.at[:, pl.ds(off, BS_SC)], sem_o[b])
        return carry

    lax.fori_loop(0, NIT // NB, body, 0)

    for b in range(NB):
        off = base + (NIT - NB + b) * BS_SC
        pltpu.make_async_copy(ob[b], out_hbm.at[:, pl.ds(off, BS_SC)],
                              sem_o[b]).wait()


def _combine_body(g_ref, s_ref, w15_ref, mT_ref, o_ref):
    h = jnp.dot(w15_ref[...], s_ref[...], preferred_element_type=jnp.float32)
    h = jnp.maximum(h, 0.0)
    p = jnp.dot(mT_ref[...], h, preferred_element_type=jnp.float32)
    o_ref[...] = g_ref[...] + p


def _combine_body_b(prev_ref, g_ref, s_ref, w15_ref, mT_ref, o_ref):
    del prev_ref
    h = jnp.dot(w15_ref[...], s_ref[...], preferred_element_type=jnp.float32)
    h = jnp.maximum(h, 0.0)
    p = jnp.dot(mT_ref[...], h, preferred_element_type=jnp.float32)
    o_ref[...] = g_ref[...] + p


def kernel(categories, popularity, cluster_ids, temporal_scores,
           geographic_scores, emb_table, W1, b1, W2, b2, Wc, bc):
    cats = categories.astype(jnp.int32)
    cats_pad = jnp.concatenate([cats, jnp.zeros((NPAD - N,), jnp.int32)])
    ones = jnp.ones((N,), jnp.float32)
    stats5 = jnp.stack([popularity, cluster_ids, temporal_scores,
                        geographic_scores, ones], axis=0)

    wcaT = Wc[:, :H].T
    wcbT = Wc[:, H:].T
    wcb = Wc[:, H:]
    b2r = b2.reshape(1, H)
    bcr = bc.reshape(1, D)
    w15 = jnp.concatenate([W1, b1.reshape(H, 1)], axis=1)

    t2p, mT = pl.pallas_call(
        _prep_body,
        out_shape=(
            jax.ShapeDtypeStruct((TP, DP), jnp.float32),
            jax.ShapeDtypeStruct((D, H), jnp.float32),
        ),
    )(emb_table[0::2], emb_table[1::2], wcaT, wcbT, wcb, W2, b2r, bcr)

    mesh = plsc.VectorSubcoreMesh(core_axis_name="c", subcore_axis_name="s")
    sc_kwargs = dict(
        mesh=mesh,
        out_type=jax.ShapeDtypeStruct((D, NPAD2), jnp.float32),
        scratch_types=[
            pltpu.VMEM((TP, DP), jnp.float32),
            pltpu.VMEM((BS_SC,), jnp.int32),
            pltpu.VMEM((BS_SC,), jnp.int32),
            pltpu.VMEM((D, BS_SC), jnp.float32),
            pltpu.VMEM((D, BS_SC), jnp.float32),
            pltpu.SemaphoreType.DMA,
            pltpu.SemaphoreType.DMA,
            pltpu.SemaphoreType.DMA,
            pltpu.SemaphoreType.DMA,
        ],
        compiler_params=pltpu.CompilerParams(use_tc_tiling_on_sc=True,
                                             needs_layout_passes=False),
    )
    g_ta = pl.kernel(_sc_gather_body, **sc_kwargs)(cats_pad[:NPAD2], t2p)
    g_tb = pl.kernel(_sc_gather_body, **sc_kwargs)(cats_pad[NPAD2:], t2p)

    out_a = pl.pallas_call(
        _combine_body,
        grid=(GRID_A,),
        in_specs=[
            pl.BlockSpec((D, BT), lambda i: (0, i)),
            pl.BlockSpec((5, BT), lambda i: (0, i)),
            pl.BlockSpec((H, 5), lambda i: (0, 0)),
            pl.BlockSpec((D, H), lambda i: (0, 0)),
        ],
        out_specs=pl.BlockSpec((D, BT), lambda i: (0, i)),
        out_shape=jax.ShapeDtypeStruct((D, N), jnp.float32),
    )(g_ta, stats5, w15, mT)

    out_t = pl.pallas_call(
        _combine_body_b,
        grid=(GRID_B,),
        in_specs=[
            pl.BlockSpec(memory_space=pl.ANY),
            pl.BlockSpec((D, BT), lambda i: (0, i)),
            pl.BlockSpec((5, BT), lambda i: (0, i + GRID_A)),
            pl.BlockSpec((H, 5), lambda i: (0, 0)),
            pl.BlockSpec((D, H), lambda i: (0, 0)),
        ],
        out_specs=pl.BlockSpec((D, BT), lambda i: (0, i + GRID_A)),
        out_shape=jax.ShapeDtypeStruct((D, N), jnp.float32),
        input_output_aliases={0: 0},
    )(out_a, g_tb, stats5, w15, mT)
    return out_t.T
